# baseline (speedup 1.0000x reference)
import functools

import jax
import jax.numpy as jnp
from jax import lax
from jax.experimental import pallas as pl
from jax.experimental.pallas import tpu as pltpu
from jax.experimental.pallas import tpu_sc as plsc

_NS = 16384
_NT = 16384
_BS = 2048
_W = 4096
_NTC = _NT // _W

_NWORK = 32
_CHUNK = _NS // _NWORK
_L = 16
_IW = 128
_NIDX = _CHUNK // _IW


def _nn_idx_body(s_ref, t_ref, idx_ref, bv_ref, bj_ref):
    j = pl.program_id(1)

    s_blk = s_ref[...]
    t_blk = t_ref[...]
    a = lax.dot_general(s_blk, t_blk, (((0,), (0,)), ((), ())),
                        preferred_element_type=jnp.float32)
    s2 = jnp.transpose(
        jnp.sum(s_blk * s_blk, axis=0, keepdims=True))
    t2c = jnp.sum(t_blk * t_blk, axis=0, keepdims=True)
    key = (s2 - 2.0 * a) + t2c

    @pl.when(j == 0)
    def _():
        bv_ref[...] = jnp.full((_BS, 1), jnp.inf, jnp.float32)
        bj_ref[...] = jnp.zeros((_BS, 1), jnp.float32)

    m_c = jnp.min(key, axis=1, keepdims=True)
    lane = lax.broadcasted_iota(jnp.int32, (1, _W), 1).astype(jnp.float32)
    jglob = lane + jnp.float32(j * _W)
    cand = jnp.where(key == m_c, jglob, jnp.float32(_NT))
    j_c = jnp.min(cand, axis=1, keepdims=True)
    upd = m_c < bv_ref[...]
    bv_ref[...] = jnp.where(upd, m_c, bv_ref[...])
    bj_ref[...] = jnp.where(upd, j_c, bj_ref[...])

    @pl.when(j == _NTC - 1)
    def _():
        idx_ref[0] = bj_ref[...].astype(jnp.int32)


def _nn_indices_tc(s_all, t2d):
    idx3 = pl.pallas_call(
        _nn_idx_body,
        grid=(_NS // _BS, _NTC),
        in_specs=[
            pl.BlockSpec((3, _BS), lambda i, j: (0, i)),
            pl.BlockSpec((3, _W), lambda i, j: (0, j)),
        ],
        out_specs=pl.BlockSpec((1, _BS, 1), lambda i, j: (i, 0, 0)),
        out_shape=jax.ShapeDtypeStruct((_NS // _BS, _BS, 1), jnp.int32),
        scratch_shapes=[
            pltpu.VMEM((_BS, 1), jnp.float32),
            pltpu.VMEM((_BS, 1), jnp.float32),
        ],
    )(s_all, t2d)
    return idx3.reshape(_NS)


def _make_sc_gather():
    mesh = plsc.VectorSubcoreMesh(core_axis_name="c", subcore_axis_name="s")

    @functools.partial(
        pl.kernel, mesh=mesh,
        compiler_params=pltpu.CompilerParams(use_tc_tiling_on_sc=False),
        out_type=[
            jax.ShapeDtypeStruct((3, _NS), jnp.float32),
            jax.ShapeDtypeStruct((_NWORK, _L), jnp.float32),
        ],
        scratch_types=[
            pltpu.VMEM((_NIDX, _IW), jnp.int32),
            pltpu.VMEM((_CHUNK,), jnp.float32),
            pltpu.VMEM((_CHUNK,), jnp.float32),
            pltpu.VMEM((_CHUNK,), jnp.float32),
            pltpu.VMEM((_CHUNK,), jnp.float32),
            pltpu.VMEM((_CHUNK,), jnp.float32),
            pltpu.VMEM((_CHUNK,), jnp.float32),
            pltpu.VMEM((_CHUNK,), jnp.float32),
            pltpu.VMEM((_CHUNK,), jnp.float32),
            pltpu.VMEM((_CHUNK,), jnp.float32),
            pltpu.VMEM((_L,), jnp.float32),
            pltpu.SemaphoreType.DMA,
        ],
    )
    def gather_pw(tx_hbm, ty_hbm, tz_hbm, sx_hbm, sy_hbm, sz_hbm, idx_hbm,
                  pw_hbm, psum_hbm,
                  idx_v, sx_v, sy_v, sz_v, gx_v, gy_v, gz_v,
                  pwx_v, pwy_v, pwz_v, acc_v, sem):
        wid = lax.axis_index("s") * 2 + lax.axis_index("c")
        base = wid * _CHUNK
        pltpu.sync_copy(idx_hbm.at[pl.ds(wid * _NIDX, _NIDX)], idx_v)
        pltpu.sync_copy(sx_hbm.at[pl.ds(base, _CHUNK)], sx_v)
        pltpu.sync_copy(sy_hbm.at[pl.ds(base, _CHUNK)], sy_v)
        pltpu.sync_copy(sz_hbm.at[pl.ds(base, _CHUNK)], sz_v)
        copies = []
        for j in range(_NIDX):
            row = idx_v.at[j]
            dst = pl.ds(j * _IW, _IW)
            copies.append(pltpu.async_copy(tx_hbm.at[row], gx_v.at[dst], sem))
            copies.append(pltpu.async_copy(ty_hbm.at[row], gy_v.at[dst], sem))
            copies.append(pltpu.async_copy(tz_hbm.at[row], gz_v.at[dst], sem))
        for c in copies:
            c.wait()
        acc = jnp.zeros((_L,), jnp.float32)
        for k in range(_CHUNK // _L):
            sl = pl.ds(k * _L, _L)
            dx = sx_v[sl] - gx_v[sl]
            dy = sy_v[sl] - gy_v[sl]
            dz = sz_v[sl] - gz_v[sl]
            px = dx * dx
            py = dy * dy
            pz = dz * dz
            pwx_v[sl] = px
            pwy_v[sl] = py
            pwz_v[sl] = pz
            acc = acc + px + py + pz
        acc_v[...] = acc
        pltpu.sync_copy(pwx_v, pw_hbm.at[0, pl.ds(base, _CHUNK)])
        pltpu.sync_copy(pwy_v, pw_hbm.at[1, pl.ds(base, _CHUNK)])
        pltpu.sync_copy(pwz_v, pw_hbm.at[2, pl.ds(base, _CHUNK)])
        pltpu.sync_copy(acc_v, psum_hbm.at[wid])

    return gather_pw


_sc_gather_cache = []


def _sc_gather(*args):
    if not _sc_gather_cache:
        _sc_gather_cache.append(_make_sc_gather())
    return _sc_gather_cache[0](*args)


def kernel(source_point_cloud_transformed, source_normal_list_transformed,
           target_point_cloud, target_normal_list,
           compute_pointwise_loss_bool):
    src = source_point_cloud_transformed
    tgt = target_point_cloud

    idx = _nn_indices_tc(src[0], tgt[0])
    idx2 = idx.reshape(_NS // _IW, _IW)

    pw2d, psum = _sc_gather(
        tgt[0, 0], tgt[0, 1], tgt[0, 2],
        src[0, 0], src[0, 1], src[0, 2], idx2)

    pw = pw2d.reshape(1, 3, _NS)
    loss = jnp.sum(psum) / jnp.float32(3 * _NS)
    src_out = jnp.transpose(src, (1, 0, 2)).reshape(1, 3, -1)
    return (loss, pw, src_out)


# speedup vs baseline: 1.0418x; 1.0418x over previous
import functools

import jax
import jax.numpy as jnp
from jax import lax
from jax.experimental import pallas as pl
from jax.experimental.pallas import tpu as pltpu
from jax.experimental.pallas import tpu_sc as plsc

_NS = 16384
_NT = 16384
_BS = 2048
_W = 4096
_NTC = _NT // _W

_NWORK = 32
_CHUNK = _NS // _NWORK
_L = 16
_IW = 128
_NIDX = _CHUNK // _IW


def _nn_idx_body(s_ref, t_ref, idx_ref, bv_ref, bj_ref):
    j = pl.program_id(1)

    s_blk = s_ref[...]
    t_blk = t_ref[...]
    a = jnp.dot(s_blk, t_blk, preferred_element_type=jnp.float32)
    s2 = jnp.sum(s_blk * s_blk, axis=1, keepdims=True)
    t2c = jnp.sum(t_blk * t_blk, axis=0, keepdims=True)
    key = (s2 - 2.0 * a) + t2c

    @pl.when(j == 0)
    def _():
        bv_ref[...] = jnp.full((_BS, 1), jnp.inf, jnp.float32)
        bj_ref[...] = jnp.zeros((_BS, 1), jnp.float32)

    m_c = jnp.min(key, axis=1, keepdims=True)
    lane = lax.broadcasted_iota(jnp.int32, (1, _W), 1).astype(jnp.float32)
    jglob = lane + jnp.float32(j * _W)
    cand = jnp.where(key == m_c, jglob, jnp.float32(_NT))
    j_c = jnp.min(cand, axis=1, keepdims=True)
    upd = m_c < bv_ref[...]
    bv_ref[...] = jnp.where(upd, m_c, bv_ref[...])
    bj_ref[...] = jnp.where(upd, j_c, bj_ref[...])

    @pl.when(j == _NTC - 1)
    def _():
        idx_ref[0] = bj_ref[...].astype(jnp.int32)


def _nn_indices_tc(s_all, t2d):
    idx3 = pl.pallas_call(
        _nn_idx_body,
        grid=(_NS // _BS, _NTC),
        in_specs=[
            pl.BlockSpec((_BS, 3), lambda i, j: (i, 0)),
            pl.BlockSpec((3, _W), lambda i, j: (0, j)),
        ],
        out_specs=pl.BlockSpec((1, _BS, 1), lambda i, j: (i, 0, 0)),
        out_shape=jax.ShapeDtypeStruct((_NS // _BS, _BS, 1), jnp.int32),
        scratch_shapes=[
            pltpu.VMEM((_BS, 1), jnp.float32),
            pltpu.VMEM((_BS, 1), jnp.float32),
        ],
    )(s_all, t2d)
    return idx3.reshape(_NS)


def _make_sc_gather():
    mesh = plsc.VectorSubcoreMesh(core_axis_name="c", subcore_axis_name="s")

    @functools.partial(
        pl.kernel, mesh=mesh,
        compiler_params=pltpu.CompilerParams(use_tc_tiling_on_sc=False),
        out_type=[
            jax.ShapeDtypeStruct((3, _NS), jnp.float32),
            jax.ShapeDtypeStruct((_NWORK, _L), jnp.float32),
        ],
        scratch_types=[
            pltpu.VMEM((_NIDX, _IW), jnp.int32),
            pltpu.VMEM((_CHUNK,), jnp.float32),
            pltpu.VMEM((_CHUNK,), jnp.float32),
            pltpu.VMEM((_CHUNK,), jnp.float32),
            pltpu.VMEM((_CHUNK,), jnp.float32),
            pltpu.VMEM((_CHUNK,), jnp.float32),
            pltpu.VMEM((_CHUNK,), jnp.float32),
            pltpu.VMEM((_CHUNK,), jnp.float32),
            pltpu.VMEM((_CHUNK,), jnp.float32),
            pltpu.VMEM((_CHUNK,), jnp.float32),
            pltpu.VMEM((_L,), jnp.float32),
            pltpu.SemaphoreType.DMA,
        ],
    )
    def gather_pw(tx_hbm, ty_hbm, tz_hbm, sx_hbm, sy_hbm, sz_hbm, idx_hbm,
                  pw_hbm, psum_hbm,
                  idx_v, sx_v, sy_v, sz_v, gx_v, gy_v, gz_v,
                  pwx_v, pwy_v, pwz_v, acc_v, sem):
        wid = lax.axis_index("s") * 2 + lax.axis_index("c")
        base = wid * _CHUNK
        pltpu.sync_copy(idx_hbm.at[pl.ds(wid * _NIDX, _NIDX)], idx_v)
        pltpu.sync_copy(sx_hbm.at[pl.ds(base, _CHUNK)], sx_v)
        pltpu.sync_copy(sy_hbm.at[pl.ds(base, _CHUNK)], sy_v)
        pltpu.sync_copy(sz_hbm.at[pl.ds(base, _CHUNK)], sz_v)
        copies = []
        for j in range(_NIDX):
            row = idx_v.at[j]
            dst = pl.ds(j * _IW, _IW)
            copies.append(pltpu.async_copy(tx_hbm.at[row], gx_v.at[dst], sem))
            copies.append(pltpu.async_copy(ty_hbm.at[row], gy_v.at[dst], sem))
            copies.append(pltpu.async_copy(tz_hbm.at[row], gz_v.at[dst], sem))
        for c in copies:
            c.wait()
        acc = jnp.zeros((_L,), jnp.float32)
        for k in range(_CHUNK // _L):
            sl = pl.ds(k * _L, _L)
            dx = sx_v[sl] - gx_v[sl]
            dy = sy_v[sl] - gy_v[sl]
            dz = sz_v[sl] - gz_v[sl]
            px = dx * dx
            py = dy * dy
            pz = dz * dz
            pwx_v[sl] = px
            pwy_v[sl] = py
            pwz_v[sl] = pz
            acc = acc + px + py + pz
        acc_v[...] = acc
        pltpu.sync_copy(pwx_v, pw_hbm.at[0, pl.ds(base, _CHUNK)])
        pltpu.sync_copy(pwy_v, pw_hbm.at[1, pl.ds(base, _CHUNK)])
        pltpu.sync_copy(pwz_v, pw_hbm.at[2, pl.ds(base, _CHUNK)])
        pltpu.sync_copy(acc_v, psum_hbm.at[wid])

    return gather_pw


_sc_gather_cache = []


def _sc_gather(*args):
    if not _sc_gather_cache:
        _sc_gather_cache.append(_make_sc_gather())
    return _sc_gather_cache[0](*args)


def kernel(source_point_cloud_transformed, source_normal_list_transformed,
           target_point_cloud, target_normal_list,
           compute_pointwise_loss_bool):
    src = source_point_cloud_transformed
    tgt = target_point_cloud

    s_all = jnp.transpose(src[0], (1, 0))
    idx = _nn_indices_tc(s_all, tgt[0])
    idx2 = idx.reshape(_NS // _IW, _IW)

    pw2d, psum = _sc_gather(
        tgt[0, 0], tgt[0, 1], tgt[0, 2],
        src[0, 0], src[0, 1], src[0, 2], idx2)

    pw = pw2d.reshape(1, 3, _NS)
    loss = jnp.sum(psum) / jnp.float32(3 * _NS)
    src_out = jnp.transpose(src, (1, 0, 2)).reshape(1, 3, -1)
    return (loss, pw, src_out)
